# baseline (speedup 1.0000x reference)
import functools

import jax
import jax.numpy as jnp
from jax.experimental import pallas as pl
from jax.experimental.pallas import tpu as pltpu

L = 2048
D = 256
N = 16
R = 16


def _log_scan(dt, u, Bm_col, A_row, reverse):
    C, T = 128, L // 128
    a3 = jnp.exp(dt * A_row).reshape(C, T, D)
    b3 = (u * Bm_col).reshape(C, T, D)
    k = 1
    while k < T:
        if reverse:
            a_sh = jnp.concatenate(
                [a3[:, k:], jnp.ones((C, k, D), jnp.float32)], axis=1)
            b_sh = jnp.concatenate(
                [b3[:, k:], jnp.zeros((C, k, D), jnp.float32)], axis=1)
        else:
            a_sh = jnp.concatenate(
                [jnp.ones((C, k, D), jnp.float32), a3[:, :-k]], axis=1)
            b_sh = jnp.concatenate(
                [jnp.zeros((C, k, D), jnp.float32), b3[:, :-k]], axis=1)
        b3 = a3 * b_sh + b3
        a3 = a3 * a_sh
        k *= 2
    if reverse:
        Ae, Be = a3[:, 0, :], b3[:, 0, :]
    else:
        Ae, Be = a3[:, T - 1, :], b3[:, T - 1, :]
    k = 1
    while k < C:
        if reverse:
            Ae_sh = jnp.concatenate(
                [Ae[k:], jnp.ones((k, D), jnp.float32)], axis=0)
            Be_sh = jnp.concatenate(
                [Be[k:], jnp.zeros((k, D), jnp.float32)], axis=0)
        else:
            Ae_sh = jnp.concatenate(
                [jnp.ones((k, D), jnp.float32), Ae[:-k]], axis=0)
            Be_sh = jnp.concatenate(
                [jnp.zeros((k, D), jnp.float32), Be[:-k]], axis=0)
        Be = Ae * Be_sh + Be
        if 2 * k < C:
            Ae = Ae * Ae_sh
        k *= 2
    if reverse:
        h_in = jnp.concatenate(
            [Be[1:], jnp.zeros((1, D), jnp.float32)], axis=0)
    else:
        h_in = jnp.concatenate(
            [jnp.zeros((1, D), jnp.float32), Be[:-1]], axis=0)
    b3 = b3 + a3 * h_in.reshape(C, 1, D)
    return b3.reshape(L, D)


def _one_direction(x, inwT, w0, w1, cb, xprojT, dtwT, dtb, alogT_ref, Dp,
                   outwT, reverse):
    xz = jnp.dot(x, inwT, preferred_element_type=jnp.float32)
    xc0 = xz[:, :D]
    z = xz[:, D:]
    if reverse:
        xn = jnp.concatenate([xc0[1:], jnp.zeros((1, D), jnp.float32)], axis=0)
    else:
        xn = jnp.concatenate([jnp.zeros((1, D), jnp.float32), xc0[:-1]], axis=0)
    xc = jax.nn.silu(xn * w0 + xc0 * w1 + cb)
    dbc = jnp.dot(xc, xprojT, preferred_element_type=jnp.float32)
    dt = jax.nn.softplus(
        jnp.dot(dbc[:, :R], dtwT, preferred_element_type=jnp.float32) + dtb)
    u = dt * xc
    lane = jax.lax.broadcasted_iota(jnp.int32, (1, R + 2 * N), 1)

    def n_step(n, y):
        A_row = -jnp.exp(alogT_ref[pl.ds(n, 1), :])
        Bcol = jnp.sum(jnp.where(lane == R + n, dbc, 0.0), axis=1,
                       keepdims=True)
        Ccol = jnp.sum(jnp.where(lane == R + N + n, dbc, 0.0), axis=1,
                       keepdims=True)
        h = _log_scan(dt, u, Bcol, A_row, reverse)
        return y + Ccol * h

    y = jax.lax.fori_loop(0, N, n_step, jnp.zeros((L, D), jnp.float32))
    y = (y + Dp * xc) * jax.nn.silu(z)
    return jnp.dot(y, outwT, preferred_element_type=jnp.float32)


def _body(x_ref,
          f_inwT, f_w0, f_w1, f_cb, f_xprojT, f_dtwT, f_dtb, f_AlogT, f_Dp,
          f_outwT,
          r_inwT, r_w0, r_w1, r_cb, r_xprojT, r_dtwT, r_dtb, r_AlogT, r_Dp,
          r_outwT,
          o_ref):
    x = x_ref[0]
    fwd = _one_direction(x, f_inwT[...], f_w0[...], f_w1[...], f_cb[...],
                         f_xprojT[...], f_dtwT[...], f_dtb[...], f_AlogT,
                         f_Dp[...], f_outwT[...], reverse=False)
    bwd = _one_direction(x, r_inwT[...], r_w0[...], r_w1[...], r_cb[...],
                         r_xprojT[...], r_dtwT[...], r_dtb[...], r_AlogT,
                         r_Dp[...], r_outwT[...], reverse=True)
    o_ref[0] = fwd + bwd


def _full(shape):
    return pl.BlockSpec(shape, lambda i: (0,) * len(shape))


@jax.jit
def kernel(x, f_in_w, f_conv_w, f_conv_b, f_xproj_w, f_dt_w, f_dt_b, f_A_log,
           f_D, f_out_w, r_in_w, r_conv_w, r_conv_b, r_xproj_w, r_dt_w, r_dt_b,
           r_A_log, r_D, r_out_w):
    B = x.shape[0]

    def prep(in_w, conv_w, conv_b, xproj_w, dt_w, dt_b, A_log, Dp, out_w):
        return (in_w.T, conv_w[:, 0][None, :], conv_w[:, 1][None, :],
                conv_b[None, :], xproj_w.T, dt_w.T, dt_b[None, :], A_log.T,
                Dp[None, :], out_w.T)

    f_args = prep(f_in_w, f_conv_w, f_conv_b, f_xproj_w, f_dt_w, f_dt_b,
                  f_A_log, f_D, f_out_w)
    r_args = prep(r_in_w, r_conv_w, r_conv_b, r_xproj_w, r_dt_w, r_dt_b,
                  r_A_log, r_D, r_out_w)
    w_args = f_args + r_args
    w_specs = [_full(w.shape) for w in w_args]

    return pl.pallas_call(
        _body,
        grid=(B,),
        in_specs=[pl.BlockSpec((1, L, D), lambda i: (i, 0, 0))] + w_specs,
        out_specs=pl.BlockSpec((1, L, D), lambda i: (i, 0, 0)),
        out_shape=jax.ShapeDtypeStruct((B, L, D), jnp.float32),
        compiler_params=pltpu.CompilerParams(
            dimension_semantics=("parallel",),
            vmem_limit_bytes=52 * 1024 * 1024,
        ),
        name="bimamba_scan",
    )(x, *w_args)


# speedup vs baseline: 1.2793x; 1.2793x over previous
import functools

import jax
import jax.numpy as jnp
from jax.experimental import pallas as pl
from jax.experimental.pallas import tpu as pltpu

L = 2048
D = 256
N = 16
R = 16


def _log_scan(dt, u, Bm_col, A_row, reverse):
    C, T = 32, L // 32
    a3 = jnp.exp(dt * A_row).reshape(C, T, D)
    b3 = (u * Bm_col).reshape(C, T, D)
    k = 1
    while k < T:
        if reverse:
            a_sh = jnp.concatenate(
                [a3[:, k:], jnp.ones((C, k, D), jnp.float32)], axis=1)
            b_sh = jnp.concatenate(
                [b3[:, k:], jnp.zeros((C, k, D), jnp.float32)], axis=1)
        else:
            a_sh = jnp.concatenate(
                [jnp.ones((C, k, D), jnp.float32), a3[:, :-k]], axis=1)
            b_sh = jnp.concatenate(
                [jnp.zeros((C, k, D), jnp.float32), b3[:, :-k]], axis=1)
        b3 = a3 * b_sh + b3
        a3 = a3 * a_sh
        k *= 2
    if reverse:
        Ae, Be = a3[:, 0, :], b3[:, 0, :]
    else:
        Ae, Be = a3[:, T - 1, :], b3[:, T - 1, :]
    k = 1
    while k < C:
        if reverse:
            Ae_sh = jnp.concatenate(
                [Ae[k:], jnp.ones((k, D), jnp.float32)], axis=0)
            Be_sh = jnp.concatenate(
                [Be[k:], jnp.zeros((k, D), jnp.float32)], axis=0)
        else:
            Ae_sh = jnp.concatenate(
                [jnp.ones((k, D), jnp.float32), Ae[:-k]], axis=0)
            Be_sh = jnp.concatenate(
                [jnp.zeros((k, D), jnp.float32), Be[:-k]], axis=0)
        Be = Ae * Be_sh + Be
        if 2 * k < C:
            Ae = Ae * Ae_sh
        k *= 2
    if reverse:
        h_in = jnp.concatenate(
            [Be[1:], jnp.zeros((1, D), jnp.float32)], axis=0)
    else:
        h_in = jnp.concatenate(
            [jnp.zeros((1, D), jnp.float32), Be[:-1]], axis=0)
    b3 = b3 + a3 * h_in.reshape(C, 1, D)
    return b3.reshape(L, D)


def _one_direction(x, inwT, w0, w1, cb, xprojT, dtwT, dtb, alogT_ref, Dp,
                   outwT, reverse):
    xz = jnp.dot(x, inwT, preferred_element_type=jnp.float32)
    xc0 = xz[:, :D]
    z = xz[:, D:]
    if reverse:
        xn = jnp.concatenate([xc0[1:], jnp.zeros((1, D), jnp.float32)], axis=0)
    else:
        xn = jnp.concatenate([jnp.zeros((1, D), jnp.float32), xc0[:-1]], axis=0)
    xc = jax.nn.silu(xn * w0 + xc0 * w1 + cb)
    dbc = jnp.dot(xc, xprojT, preferred_element_type=jnp.float32)
    dt = jax.nn.softplus(
        jnp.dot(dbc[:, :R], dtwT, preferred_element_type=jnp.float32) + dtb)
    u = dt * xc
    lane = jax.lax.broadcasted_iota(jnp.int32, (1, R + 2 * N), 1)

    def n_step(n, y):
        A_row = -jnp.exp(alogT_ref[pl.ds(n, 1), :])
        Bcol = jnp.sum(jnp.where(lane == R + n, dbc, 0.0), axis=1,
                       keepdims=True)
        Ccol = jnp.sum(jnp.where(lane == R + N + n, dbc, 0.0), axis=1,
                       keepdims=True)
        h = _log_scan(dt, u, Bcol, A_row, reverse)
        return y + Ccol * h

    y = jax.lax.fori_loop(0, N, n_step, jnp.zeros((L, D), jnp.float32))
    y = (y + Dp * xc) * jax.nn.silu(z)
    return jnp.dot(y, outwT, preferred_element_type=jnp.float32)


def _body(x_ref,
          f_inwT, f_w0, f_w1, f_cb, f_xprojT, f_dtwT, f_dtb, f_AlogT, f_Dp,
          f_outwT,
          r_inwT, r_w0, r_w1, r_cb, r_xprojT, r_dtwT, r_dtb, r_AlogT, r_Dp,
          r_outwT,
          o_ref):
    x = x_ref[0]
    fwd = _one_direction(x, f_inwT[...], f_w0[...], f_w1[...], f_cb[...],
                         f_xprojT[...], f_dtwT[...], f_dtb[...], f_AlogT,
                         f_Dp[...], f_outwT[...], reverse=False)
    bwd = _one_direction(x, r_inwT[...], r_w0[...], r_w1[...], r_cb[...],
                         r_xprojT[...], r_dtwT[...], r_dtb[...], r_AlogT,
                         r_Dp[...], r_outwT[...], reverse=True)
    o_ref[0] = fwd + bwd


def _full(shape):
    return pl.BlockSpec(shape, lambda i: (0,) * len(shape))


@jax.jit
def kernel(x, f_in_w, f_conv_w, f_conv_b, f_xproj_w, f_dt_w, f_dt_b, f_A_log,
           f_D, f_out_w, r_in_w, r_conv_w, r_conv_b, r_xproj_w, r_dt_w, r_dt_b,
           r_A_log, r_D, r_out_w):
    B = x.shape[0]

    def prep(in_w, conv_w, conv_b, xproj_w, dt_w, dt_b, A_log, Dp, out_w):
        return (in_w.T, conv_w[:, 0][None, :], conv_w[:, 1][None, :],
                conv_b[None, :], xproj_w.T, dt_w.T, dt_b[None, :], A_log.T,
                Dp[None, :], out_w.T)

    f_args = prep(f_in_w, f_conv_w, f_conv_b, f_xproj_w, f_dt_w, f_dt_b,
                  f_A_log, f_D, f_out_w)
    r_args = prep(r_in_w, r_conv_w, r_conv_b, r_xproj_w, r_dt_w, r_dt_b,
                  r_A_log, r_D, r_out_w)
    w_args = f_args + r_args
    w_specs = [_full(w.shape) for w in w_args]

    return pl.pallas_call(
        _body,
        grid=(B,),
        in_specs=[pl.BlockSpec((1, L, D), lambda i: (i, 0, 0))] + w_specs,
        out_specs=pl.BlockSpec((1, L, D), lambda i: (i, 0, 0)),
        out_shape=jax.ShapeDtypeStruct((B, L, D), jnp.float32),
        compiler_params=pltpu.CompilerParams(
            dimension_semantics=("parallel",),
            vmem_limit_bytes=52 * 1024 * 1024,
        ),
        name="bimamba_scan",
    )(x, *w_args)


# speedup vs baseline: 1.3380x; 1.0459x over previous
import functools

import jax
import jax.numpy as jnp
from jax.experimental import pallas as pl
from jax.experimental.pallas import tpu as pltpu

L = 2048
D = 256
N = 16
R = 16


def _log_scan(a, b, reverse):
    W = a.shape[1]
    C, T = 32, L // 32
    a3 = a.reshape(C, T, W)
    b3 = b.reshape(C, T, W)
    D = W
    k = 1
    while k < T:
        if reverse:
            a_sh = jnp.concatenate(
                [a3[:, k:], jnp.ones((C, k, D), jnp.float32)], axis=1)
            b_sh = jnp.concatenate(
                [b3[:, k:], jnp.zeros((C, k, D), jnp.float32)], axis=1)
        else:
            a_sh = jnp.concatenate(
                [jnp.ones((C, k, D), jnp.float32), a3[:, :-k]], axis=1)
            b_sh = jnp.concatenate(
                [jnp.zeros((C, k, D), jnp.float32), b3[:, :-k]], axis=1)
        b3 = a3 * b_sh + b3
        a3 = a3 * a_sh
        k *= 2
    if reverse:
        Ae, Be = a3[:, 0, :], b3[:, 0, :]
    else:
        Ae, Be = a3[:, T - 1, :], b3[:, T - 1, :]
    k = 1
    while k < C:
        if reverse:
            Ae_sh = jnp.concatenate(
                [Ae[k:], jnp.ones((k, D), jnp.float32)], axis=0)
            Be_sh = jnp.concatenate(
                [Be[k:], jnp.zeros((k, D), jnp.float32)], axis=0)
        else:
            Ae_sh = jnp.concatenate(
                [jnp.ones((k, D), jnp.float32), Ae[:-k]], axis=0)
            Be_sh = jnp.concatenate(
                [jnp.zeros((k, D), jnp.float32), Be[:-k]], axis=0)
        Be = Ae * Be_sh + Be
        if 2 * k < C:
            Ae = Ae * Ae_sh
        k *= 2
    if reverse:
        h_in = jnp.concatenate(
            [Be[1:], jnp.zeros((1, D), jnp.float32)], axis=0)
    else:
        h_in = jnp.concatenate(
            [jnp.zeros((1, D), jnp.float32), Be[:-1]], axis=0)
    b3 = b3 + a3 * h_in.reshape(C, 1, D)
    return b3.reshape(L, D)


def _one_direction(x, inwT, w0, w1, cb, xprojT, dtwT, dtb, alogT_ref, Dp,
                   outwT, reverse):
    xz = jnp.dot(x, inwT, preferred_element_type=jnp.float32)
    xc0 = xz[:, :D]
    z = xz[:, D:]
    if reverse:
        xn = jnp.concatenate([xc0[1:], jnp.zeros((1, D), jnp.float32)], axis=0)
    else:
        xn = jnp.concatenate([jnp.zeros((1, D), jnp.float32), xc0[:-1]], axis=0)
    xc = jax.nn.silu(xn * w0 + xc0 * w1 + cb)
    dbc = jnp.dot(xc, xprojT, preferred_element_type=jnp.float32)
    dt = jax.nn.softplus(
        jnp.dot(dbc[:, :R], dtwT, preferred_element_type=jnp.float32) + dtb)
    u = dt * xc
    u2 = jnp.concatenate([u, u], axis=1)
    dt2 = jnp.concatenate([dt, dt], axis=1)
    row48 = jax.lax.broadcasted_iota(jnp.int32, (R + 2 * N, 2 * D), 0)
    half = jax.lax.broadcasted_iota(jnp.int32, (R + 2 * N, 2 * D), 1) < D

    def n_step(n, y):
        tgt = jnp.where(half, R + n, R + n + N // 2)
        sel_b = (row48 == tgt).astype(jnp.float32)
        sel_c = (row48 == tgt + N).astype(jnp.float32)
        Bbc = jnp.dot(dbc, sel_b, preferred_element_type=jnp.float32)
        Cbc = jnp.dot(dbc, sel_c, preferred_element_type=jnp.float32)
        A2 = -jnp.exp(jnp.concatenate(
            [alogT_ref[pl.ds(n, 1), :],
             alogT_ref[pl.ds(n + N // 2, 1), :]], axis=1))
        h = _log_scan(jnp.exp(dt2 * A2), u2 * Bbc, reverse)
        w = Cbc * h
        return y + w[:, :D] + w[:, D:]

    y = jax.lax.fori_loop(0, N // 2, n_step,
                          jnp.zeros((L, D), jnp.float32))
    y = (y + Dp * xc) * jax.nn.silu(z)
    return jnp.dot(y, outwT, preferred_element_type=jnp.float32)


def _body(x_ref,
          f_inwT, f_w0, f_w1, f_cb, f_xprojT, f_dtwT, f_dtb, f_AlogT, f_Dp,
          f_outwT,
          r_inwT, r_w0, r_w1, r_cb, r_xprojT, r_dtwT, r_dtb, r_AlogT, r_Dp,
          r_outwT,
          o_ref):
    x = x_ref[0]
    fwd = _one_direction(x, f_inwT[...], f_w0[...], f_w1[...], f_cb[...],
                         f_xprojT[...], f_dtwT[...], f_dtb[...], f_AlogT,
                         f_Dp[...], f_outwT[...], reverse=False)
    bwd = _one_direction(x, r_inwT[...], r_w0[...], r_w1[...], r_cb[...],
                         r_xprojT[...], r_dtwT[...], r_dtb[...], r_AlogT,
                         r_Dp[...], r_outwT[...], reverse=True)
    o_ref[0] = fwd + bwd


def _full(shape):
    return pl.BlockSpec(shape, lambda i: (0,) * len(shape))


@jax.jit
def kernel(x, f_in_w, f_conv_w, f_conv_b, f_xproj_w, f_dt_w, f_dt_b, f_A_log,
           f_D, f_out_w, r_in_w, r_conv_w, r_conv_b, r_xproj_w, r_dt_w, r_dt_b,
           r_A_log, r_D, r_out_w):
    B = x.shape[0]

    def prep(in_w, conv_w, conv_b, xproj_w, dt_w, dt_b, A_log, Dp, out_w):
        return (in_w.T, conv_w[:, 0][None, :], conv_w[:, 1][None, :],
                conv_b[None, :], xproj_w.T, dt_w.T, dt_b[None, :], A_log.T,
                Dp[None, :], out_w.T)

    f_args = prep(f_in_w, f_conv_w, f_conv_b, f_xproj_w, f_dt_w, f_dt_b,
                  f_A_log, f_D, f_out_w)
    r_args = prep(r_in_w, r_conv_w, r_conv_b, r_xproj_w, r_dt_w, r_dt_b,
                  r_A_log, r_D, r_out_w)
    w_args = f_args + r_args
    w_specs = [_full(w.shape) for w in w_args]

    return pl.pallas_call(
        _body,
        grid=(B,),
        in_specs=[pl.BlockSpec((1, L, D), lambda i: (i, 0, 0))] + w_specs,
        out_specs=pl.BlockSpec((1, L, D), lambda i: (i, 0, 0)),
        out_shape=jax.ShapeDtypeStruct((B, L, D), jnp.float32),
        compiler_params=pltpu.CompilerParams(
            dimension_semantics=("parallel",),
            vmem_limit_bytes=52 * 1024 * 1024,
        ),
        name="bimamba_scan",
    )(x, *w_args)


# speedup vs baseline: 1.3955x; 1.0429x over previous
import functools

import jax
import jax.numpy as jnp
from jax.experimental import pallas as pl
from jax.experimental.pallas import tpu as pltpu

L = 2048
D = 256
N = 16
R = 16


def _log_scan(a, b, reverse):
    W = a.shape[1]
    C, T = 32, L // 32
    a3 = a.reshape(C, T, W)
    b3 = b.reshape(C, T, W)
    D = W
    k = 1
    while k < T:
        if reverse:
            ub = a3[:, :T - k] * b3[:, k:] + b3[:, :T - k]
            ua = a3[:, :T - k] * a3[:, k:]
            b3 = jnp.concatenate([ub, b3[:, T - k:]], axis=1)
            a3 = jnp.concatenate([ua, a3[:, T - k:]], axis=1)
        else:
            ub = a3[:, k:] * b3[:, :T - k] + b3[:, k:]
            ua = a3[:, k:] * a3[:, :T - k]
            b3 = jnp.concatenate([b3[:, :k], ub], axis=1)
            a3 = jnp.concatenate([a3[:, :k], ua], axis=1)
        k *= 2
    if reverse:
        Ae, Be = a3[:, 0, :], b3[:, 0, :]
    else:
        Ae, Be = a3[:, T - 1, :], b3[:, T - 1, :]
    k = 1
    while k < C:
        if reverse:
            uB = Ae[:C - k] * Be[k:] + Be[:C - k]
            Be = jnp.concatenate([uB, Be[C - k:]], axis=0)
            if 2 * k < C:
                uA = Ae[:C - k] * Ae[k:]
                Ae = jnp.concatenate([uA, Ae[C - k:]], axis=0)
        else:
            uB = Ae[k:] * Be[:C - k] + Be[k:]
            Be = jnp.concatenate([Be[:k], uB], axis=0)
            if 2 * k < C:
                uA = Ae[k:] * Ae[:C - k]
                Ae = jnp.concatenate([Ae[:k], uA], axis=0)
        k *= 2
    if reverse:
        h_in = jnp.concatenate(
            [Be[1:], jnp.zeros((1, D), jnp.float32)], axis=0)
    else:
        h_in = jnp.concatenate(
            [jnp.zeros((1, D), jnp.float32), Be[:-1]], axis=0)
    b3 = b3 + a3 * h_in.reshape(C, 1, D)
    return b3.reshape(L, D)


def _one_direction(x, inwT, w0, w1, cb, xprojT, dtwT, dtb, alogT_ref, Dp,
                   outwT, reverse):
    xz = jnp.dot(x, inwT, preferred_element_type=jnp.float32)
    xc0 = xz[:, :D]
    z = xz[:, D:]
    if reverse:
        xn = jnp.concatenate([xc0[1:], jnp.zeros((1, D), jnp.float32)], axis=0)
    else:
        xn = jnp.concatenate([jnp.zeros((1, D), jnp.float32), xc0[:-1]], axis=0)
    xc = jax.nn.silu(xn * w0 + xc0 * w1 + cb)
    dbc = jnp.dot(xc, xprojT, preferred_element_type=jnp.float32)
    dt = jax.nn.softplus(
        jnp.dot(dbc[:, :R], dtwT, preferred_element_type=jnp.float32) + dtb)
    u = dt * xc
    u2 = jnp.concatenate([u, u], axis=1)
    dt2 = jnp.concatenate([dt, dt], axis=1)
    row48 = jax.lax.broadcasted_iota(jnp.int32, (R + 2 * N, 2 * D), 0)
    half = jax.lax.broadcasted_iota(jnp.int32, (R + 2 * N, 2 * D), 1) < D

    def n_step(n, y):
        tgt = jnp.where(half, R + n, R + n + N // 2)
        sel_b = (row48 == tgt).astype(jnp.float32)
        sel_c = (row48 == tgt + N).astype(jnp.float32)
        Bbc = jnp.dot(dbc, sel_b, preferred_element_type=jnp.float32)
        Cbc = jnp.dot(dbc, sel_c, preferred_element_type=jnp.float32)
        A2 = -jnp.exp(jnp.concatenate(
            [alogT_ref[pl.ds(n, 1), :],
             alogT_ref[pl.ds(n + N // 2, 1), :]], axis=1))
        h = _log_scan(jnp.exp(dt2 * A2), u2 * Bbc, reverse)
        w = Cbc * h
        return y + w[:, :D] + w[:, D:]

    y = jax.lax.fori_loop(0, N // 2, n_step,
                          jnp.zeros((L, D), jnp.float32))
    y = (y + Dp * xc) * jax.nn.silu(z)
    return jnp.dot(y, outwT, preferred_element_type=jnp.float32)


def _body(x_ref,
          f_inwT, f_w0, f_w1, f_cb, f_xprojT, f_dtwT, f_dtb, f_AlogT, f_Dp,
          f_outwT,
          r_inwT, r_w0, r_w1, r_cb, r_xprojT, r_dtwT, r_dtb, r_AlogT, r_Dp,
          r_outwT,
          o_ref):
    x = x_ref[0]
    fwd = _one_direction(x, f_inwT[...], f_w0[...], f_w1[...], f_cb[...],
                         f_xprojT[...], f_dtwT[...], f_dtb[...], f_AlogT,
                         f_Dp[...], f_outwT[...], reverse=False)
    bwd = _one_direction(x, r_inwT[...], r_w0[...], r_w1[...], r_cb[...],
                         r_xprojT[...], r_dtwT[...], r_dtb[...], r_AlogT,
                         r_Dp[...], r_outwT[...], reverse=True)
    o_ref[0] = fwd + bwd


def _full(shape):
    return pl.BlockSpec(shape, lambda i: (0,) * len(shape))


@jax.jit
def kernel(x, f_in_w, f_conv_w, f_conv_b, f_xproj_w, f_dt_w, f_dt_b, f_A_log,
           f_D, f_out_w, r_in_w, r_conv_w, r_conv_b, r_xproj_w, r_dt_w, r_dt_b,
           r_A_log, r_D, r_out_w):
    B = x.shape[0]

    def prep(in_w, conv_w, conv_b, xproj_w, dt_w, dt_b, A_log, Dp, out_w):
        return (in_w.T, conv_w[:, 0][None, :], conv_w[:, 1][None, :],
                conv_b[None, :], xproj_w.T, dt_w.T, dt_b[None, :], A_log.T,
                Dp[None, :], out_w.T)

    f_args = prep(f_in_w, f_conv_w, f_conv_b, f_xproj_w, f_dt_w, f_dt_b,
                  f_A_log, f_D, f_out_w)
    r_args = prep(r_in_w, r_conv_w, r_conv_b, r_xproj_w, r_dt_w, r_dt_b,
                  r_A_log, r_D, r_out_w)
    w_args = f_args + r_args
    w_specs = [_full(w.shape) for w in w_args]

    return pl.pallas_call(
        _body,
        grid=(B,),
        in_specs=[pl.BlockSpec((1, L, D), lambda i: (i, 0, 0))] + w_specs,
        out_specs=pl.BlockSpec((1, L, D), lambda i: (i, 0, 0)),
        out_shape=jax.ShapeDtypeStruct((B, L, D), jnp.float32),
        compiler_params=pltpu.CompilerParams(
            dimension_semantics=("parallel",),
            vmem_limit_bytes=52 * 1024 * 1024,
        ),
        name="bimamba_scan",
    )(x, *w_args)


# speedup vs baseline: 1.5129x; 1.0841x over previous
import functools

import jax
import jax.numpy as jnp
from jax.experimental import pallas as pl
from jax.experimental.pallas import tpu as pltpu

L = 2048
D = 256
N = 16
R = 16


def _log_scan(a, b, reverse):
    W = a.shape[1]
    C, T = 64, L // 64
    a3 = a.reshape(C, T, W)
    b3 = b.reshape(C, T, W)
    D = W
    k = 1
    while k < T:
        if reverse:
            ub = a3[:, :T - k] * b3[:, k:] + b3[:, :T - k]
            ua = a3[:, :T - k] * a3[:, k:]
            b3 = jnp.concatenate([ub, b3[:, T - k:]], axis=1)
            a3 = jnp.concatenate([ua, a3[:, T - k:]], axis=1)
        else:
            ub = a3[:, k:] * b3[:, :T - k] + b3[:, k:]
            ua = a3[:, k:] * a3[:, :T - k]
            b3 = jnp.concatenate([b3[:, :k], ub], axis=1)
            a3 = jnp.concatenate([a3[:, :k], ua], axis=1)
        k *= 2
    if reverse:
        Ae, Be = a3[:, 0, :], b3[:, 0, :]
    else:
        Ae, Be = a3[:, T - 1, :], b3[:, T - 1, :]
    k = 1
    while k < C:
        if reverse:
            uB = Ae[:C - k] * Be[k:] + Be[:C - k]
            Be = jnp.concatenate([uB, Be[C - k:]], axis=0)
            if 2 * k < C:
                uA = Ae[:C - k] * Ae[k:]
                Ae = jnp.concatenate([uA, Ae[C - k:]], axis=0)
        else:
            uB = Ae[k:] * Be[:C - k] + Be[k:]
            Be = jnp.concatenate([Be[:k], uB], axis=0)
            if 2 * k < C:
                uA = Ae[k:] * Ae[:C - k]
                Ae = jnp.concatenate([Ae[:k], uA], axis=0)
        k *= 2
    if reverse:
        h_in = jnp.concatenate(
            [Be[1:], jnp.zeros((1, D), jnp.float32)], axis=0)
    else:
        h_in = jnp.concatenate(
            [jnp.zeros((1, D), jnp.float32), Be[:-1]], axis=0)
    b3 = b3 + a3 * h_in.reshape(C, 1, D)
    return b3.reshape(L, D)


def _one_direction(x, inwT, w0, w1, cb, xprojT, dtwT, dtb, alogT_ref, Dp,
                   outwT, reverse):
    xz = jnp.dot(x, inwT, preferred_element_type=jnp.float32)
    xc0 = xz[:, :D]
    z = xz[:, D:]
    if reverse:
        xn = jnp.concatenate([xc0[1:], jnp.zeros((1, D), jnp.float32)], axis=0)
    else:
        xn = jnp.concatenate([jnp.zeros((1, D), jnp.float32), xc0[:-1]], axis=0)
    xc = jax.nn.silu(xn * w0 + xc0 * w1 + cb)
    dbc = jnp.dot(xc, xprojT, preferred_element_type=jnp.float32)
    dt = jax.nn.softplus(
        jnp.dot(dbc[:, :R], dtwT, preferred_element_type=jnp.float32) + dtb)
    u = dt * xc
    u2 = jnp.concatenate([u, u], axis=1)
    dt2 = jnp.concatenate([dt, dt], axis=1)
    row48 = jax.lax.broadcasted_iota(jnp.int32, (R + 2 * N, 2 * D), 0)
    half = jax.lax.broadcasted_iota(jnp.int32, (R + 2 * N, 2 * D), 1) < D

    def n_step(n, y):
        tgt = jnp.where(half, R + n, R + n + N // 2)
        sel_b = (row48 == tgt).astype(jnp.float32)
        sel_c = (row48 == tgt + N).astype(jnp.float32)
        Bbc = jnp.dot(dbc, sel_b, preferred_element_type=jnp.float32)
        Cbc = jnp.dot(dbc, sel_c, preferred_element_type=jnp.float32)
        A2 = -jnp.exp(jnp.concatenate(
            [alogT_ref[pl.ds(n, 1), :],
             alogT_ref[pl.ds(n + N // 2, 1), :]], axis=1))
        h = _log_scan(jnp.exp(dt2 * A2), u2 * Bbc, reverse)
        w = Cbc * h
        return y + w[:, :D] + w[:, D:]

    y = jax.lax.fori_loop(0, N // 2, n_step,
                          jnp.zeros((L, D), jnp.float32))
    y = (y + Dp * xc) * jax.nn.silu(z)
    return jnp.dot(y, outwT, preferred_element_type=jnp.float32)


def _body(x_ref,
          f_inwT, f_w0, f_w1, f_cb, f_xprojT, f_dtwT, f_dtb, f_AlogT, f_Dp,
          f_outwT,
          r_inwT, r_w0, r_w1, r_cb, r_xprojT, r_dtwT, r_dtb, r_AlogT, r_Dp,
          r_outwT,
          o_ref):
    x = x_ref[0]
    fwd = _one_direction(x, f_inwT[...], f_w0[...], f_w1[...], f_cb[...],
                         f_xprojT[...], f_dtwT[...], f_dtb[...], f_AlogT,
                         f_Dp[...], f_outwT[...], reverse=False)
    bwd = _one_direction(x, r_inwT[...], r_w0[...], r_w1[...], r_cb[...],
                         r_xprojT[...], r_dtwT[...], r_dtb[...], r_AlogT,
                         r_Dp[...], r_outwT[...], reverse=True)
    o_ref[0] = fwd + bwd


def _full(shape):
    return pl.BlockSpec(shape, lambda i: (0,) * len(shape))


@jax.jit
def kernel(x, f_in_w, f_conv_w, f_conv_b, f_xproj_w, f_dt_w, f_dt_b, f_A_log,
           f_D, f_out_w, r_in_w, r_conv_w, r_conv_b, r_xproj_w, r_dt_w, r_dt_b,
           r_A_log, r_D, r_out_w):
    B = x.shape[0]

    def prep(in_w, conv_w, conv_b, xproj_w, dt_w, dt_b, A_log, Dp, out_w):
        return (in_w.T, conv_w[:, 0][None, :], conv_w[:, 1][None, :],
                conv_b[None, :], xproj_w.T, dt_w.T, dt_b[None, :], A_log.T,
                Dp[None, :], out_w.T)

    f_args = prep(f_in_w, f_conv_w, f_conv_b, f_xproj_w, f_dt_w, f_dt_b,
                  f_A_log, f_D, f_out_w)
    r_args = prep(r_in_w, r_conv_w, r_conv_b, r_xproj_w, r_dt_w, r_dt_b,
                  r_A_log, r_D, r_out_w)
    w_args = f_args + r_args
    w_specs = [_full(w.shape) for w in w_args]

    return pl.pallas_call(
        _body,
        grid=(B,),
        in_specs=[pl.BlockSpec((1, L, D), lambda i: (i, 0, 0))] + w_specs,
        out_specs=pl.BlockSpec((1, L, D), lambda i: (i, 0, 0)),
        out_shape=jax.ShapeDtypeStruct((B, L, D), jnp.float32),
        compiler_params=pltpu.CompilerParams(
            dimension_semantics=("parallel",),
            vmem_limit_bytes=52 * 1024 * 1024,
        ),
        name="bimamba_scan",
    )(x, *w_args)
